# baseline (speedup 1.0000x reference)
import functools

import jax
import jax.numpy as jnp
from jax import lax
from jax.experimental import pallas as pl
from jax.experimental.pallas import tpu as pltpu
from jax.experimental.pallas import tpu_sc as plsc

NC = 2
NS = 16
CH = 128


def _edge_pass_builder(N_pad, D, TOTALC, K0, K1, SB):
    mesh = plsc.VectorSubcoreMesh(core_axis_name="c", subcore_axis_name="s")
    f32 = jnp.float32
    KMAX = max(K0, K1)

    @functools.partial(
        pl.kernel,
        out_type=[
            jax.ShapeDtypeStruct((N_pad, D), f32),
            jax.ShapeDtypeStruct((N_pad, D), f32),
        ],
        mesh=mesh,
        scratch_types=[
            pltpu.VMEM((KMAX, CH), jnp.int32),
            pltpu.VMEM((KMAX, CH), jnp.int32),
            pltpu.VMEM((CH, D), f32),
            pltpu.VMEM_SHARED((N_pad, D), f32),
            pltpu.SemaphoreType.DMA,
        ],
    )
    def edge_pass(h_hbm, src_hbm, dst_hbm, zero_hbm, out0_hbm, out1_hbm,
                  src_v, dst_v, rows_a, agg_sh, sem_a):
        c = lax.axis_index("c")
        s = lax.axis_index("s")

        @pl.when(s == 0)
        def _():
            pltpu.sync_copy(zero_hbm, agg_sh)

        plsc.subcore_barrier()

        k_tile = jnp.where(c == 0, K0, K1)
        base = jnp.where(c == 0, s * K0, 16 * K0 + s * K1)

        pltpu.sync_copy(src_hbm.at[pl.ds(base, KMAX)], src_v)
        pltpu.sync_copy(dst_hbm.at[pl.ds(base, KMAX)], dst_v)

        def body(j, carry):
            pltpu.async_copy(h_hbm.at[src_v.at[j]], rows_a, sem_a).wait()
            pltpu.sync_copy(rows_a, agg_sh.at[dst_v.at[j]], add=True)
            return carry

        lax.fori_loop(0, k_tile, body, 0)

        plsc.subcore_barrier()

        @pl.when((s == 0) & (c == 0))
        def _():
            pltpu.sync_copy(agg_sh, out0_hbm)

        @pl.when((s == 0) & (c == 1))
        def _():
            pltpu.sync_copy(agg_sh, out1_hbm)

    return edge_pass


def _deg_builder(N_pad, W, NCHUNK):
    mesh = plsc.VectorSubcoreMesh(core_axis_name="c", subcore_axis_name="s")
    f32 = jnp.float32

    @functools.partial(
        pl.kernel,
        out_type=[
            jax.ShapeDtypeStruct((N_pad, W), f32),
            jax.ShapeDtypeStruct((N_pad, W), f32),
        ],
        mesh=mesh,
        scratch_types=[
            pltpu.VMEM((NCHUNK, CH), jnp.int32),
            pltpu.VMEM((CH, W), f32),
            pltpu.VMEM_SHARED((N_pad, W), f32),
        ],
    )
    def deg_pass(dst_hbm, ones_hbm, zero_hbm, out0_hbm, out1_hbm,
                 dst_v, ones_v, deg_sh):
        c = lax.axis_index("c")
        s = lax.axis_index("s")
        wid = c * NS + s

        @pl.when(s == 0)
        def _():
            pltpu.sync_copy(zero_hbm, deg_sh)

        plsc.subcore_barrier()

        pltpu.sync_copy(dst_hbm.at[wid], dst_v)
        pltpu.sync_copy(ones_hbm, ones_v)

        def body(j, carry):
            pltpu.sync_copy(ones_v, deg_sh.at[dst_v.at[j]], add=True)
            return carry

        lax.fori_loop(0, NCHUNK, body, 0)

        plsc.subcore_barrier()

        @pl.when((s == 0) & (c == 0))
        def _():
            pltpu.sync_copy(deg_sh, out0_hbm)

        @pl.when((s == 0) & (c == 1))
        def _():
            pltpu.sync_copy(deg_sh, out1_hbm)

    return deg_pass


def kernel(x, W_enc, b_enc, W_layers, b_layers, gammas, betas, W_out, b_out,
           edge_index):
    N, D = x.shape
    L = W_layers.shape[0]
    E = edge_index.shape[1]
    NW = NC * NS
    f32 = jnp.float32

    NCHUNK = -(-E // (NW * CH))
    NCHUNK = -(-NCHUNK // 4) * 4
    E_pad = NW * NCHUNK * CH
    N_pad = N + 16

    src = edge_index[0]
    dst = edge_index[1]
    pad = E_pad - E
    src_p = jnp.concatenate([src, jnp.zeros((pad,), jnp.int32)])
    dst_p = jnp.concatenate([dst, jnp.full((pad,), N, jnp.int32)])
    dst3 = dst_p.reshape(NW, NCHUNK, CH)

    zeroND = jnp.zeros((N_pad, D), f32)
    onesW = jnp.ones((CH, D), f32)

    TOTALC = NW * NCHUNK
    K_pair = TOTALC // 16
    K0 = K_pair // 2
    K1 = K_pair - K0
    KMAX = max(K0, K1)
    tail = jnp.zeros((KMAX * CH,), jnp.int32)
    src2 = jnp.concatenate([src_p, tail]).reshape(TOTALC + KMAX, CH)
    dst2 = jnp.concatenate([dst_p, jnp.full((KMAX * CH,), N, jnp.int32)]
                           ).reshape(TOTALC + KMAX, CH)

    edge_pass = _edge_pass_builder(N_pad, D, TOTALC, K0, K1, 0)
    deg_pass = _deg_builder(N_pad, D, NCHUNK)

    RB = 2000 if N % 2000 == 0 else (1000 if N % 1000 == 0 else 8)
    G = N // RB

    def _enc_body(x_ref, w_ref, b_ref, o_ref):
        o_ref[...] = jnp.dot(x_ref[...], w_ref[...],
                             preferred_element_type=f32) + b_ref[...]

    h = pl.pallas_call(
        _enc_body,
        grid=(G,),
        in_specs=[
            pl.BlockSpec((RB, D), lambda i: (i, 0)),
            pl.BlockSpec((D, D), lambda i: (0, 0)),
            pl.BlockSpec((1, D), lambda i: (0, 0)),
        ],
        out_specs=pl.BlockSpec((RB, D), lambda i: (i, 0)),
        out_shape=jax.ShapeDtypeStruct((N, D), f32),
    )(x, W_enc, b_enc[None])

    deg0, deg1 = deg_pass(dst3, onesW, zeroND)

    def _mm_body(p0_ref, p1_ref, d0_ref, d1_ref, w_ref, b_ref,
                 pre_ref, st_ref):
        i = pl.program_id(0)
        deg = jnp.maximum(d0_ref[...][:, :1] + d1_ref[...][:, :1], 1.0)
        agg = (p0_ref[...] + p1_ref[...]) / deg
        pre = jnp.dot(agg, w_ref[...], preferred_element_type=f32) + b_ref[...]
        pre_ref[...] = pre
        s1 = jnp.sum(pre, axis=0, keepdims=True)
        s2 = jnp.sum(pre * pre, axis=0, keepdims=True)
        upd = jnp.concatenate([s1, s2, jnp.zeros((6, pre.shape[1]), f32)],
                              axis=0)
        st_ref[...] = jnp.where(i == 0, upd, st_ref[...] + upd)

    def _bn_body(h_ref, pre_ref, st_ref, g_ref, bt_ref, o_ref):
        mu = st_ref[0:1, :] / float(N)
        ex2 = st_ref[1:2, :] / float(N)
        var = ex2 - mu * mu
        xn = (pre_ref[...] - mu) * lax.rsqrt(var + 1e-5) * g_ref[...] \
            + bt_ref[...]
        o_ref[...] = h_ref[...] + jnp.maximum(xn, 0.0)

    for l in range(L):
        part0, part1 = edge_pass(h, src2, dst2, zeroND)
        pre, stats = pl.pallas_call(
            _mm_body,
            grid=(G,),
            in_specs=[
                pl.BlockSpec((RB, D), lambda i: (i, 0)),
                pl.BlockSpec((RB, D), lambda i: (i, 0)),
                pl.BlockSpec((RB, D), lambda i: (i, 0)),
                pl.BlockSpec((RB, D), lambda i: (i, 0)),
                pl.BlockSpec((D, D), lambda i: (0, 0)),
                pl.BlockSpec((1, D), lambda i: (0, 0)),
            ],
            out_specs=[
                pl.BlockSpec((RB, D), lambda i: (i, 0)),
                pl.BlockSpec((8, D), lambda i: (0, 0)),
            ],
            out_shape=[
                jax.ShapeDtypeStruct((N, D), f32),
                jax.ShapeDtypeStruct((8, D), f32),
            ],
        )(part0, part1, deg0, deg1, W_layers[l], b_layers[l][None])

        h = pl.pallas_call(
            _bn_body,
            grid=(G,),
            in_specs=[
                pl.BlockSpec((RB, D), lambda i: (i, 0)),
                pl.BlockSpec((RB, D), lambda i: (i, 0)),
                pl.BlockSpec((8, D), lambda i: (0, 0)),
                pl.BlockSpec((1, D), lambda i: (0, 0)),
                pl.BlockSpec((1, D), lambda i: (0, 0)),
            ],
            out_specs=pl.BlockSpec((RB, D), lambda i: (i, 0)),
            out_shape=jax.ShapeDtypeStruct((N, D), f32),
        )(h, pre, stats, gammas[l][None], betas[l][None])

    NCL = b_out.shape[0]
    W_out_p = jnp.pad(W_out, ((0, 0), (0, D - NCL)))
    b_out_p = jnp.pad(b_out, (0, D - NCL))[None]

    def _out_body(h_ref, w_ref, b_ref, o_ref, acc_ref):
        i = pl.program_id(0)
        s = jnp.sum(h_ref[...], axis=0, keepdims=True)
        acc_ref[...] = jnp.where(i == 0, s, acc_ref[...] + s)

        @pl.when(i == pl.num_programs(0) - 1)
        def _():
            hg = acc_ref[...] / float(N)
            o_ref[...] = jnp.dot(hg, w_ref[...],
                                 preferred_element_type=f32) + b_ref[...]

    out = pl.pallas_call(
        _out_body,
        grid=(G,),
        in_specs=[
            pl.BlockSpec((RB, D), lambda i: (i, 0)),
            pl.BlockSpec((D, D), lambda i: (0, 0)),
            pl.BlockSpec((1, D), lambda i: (0, 0)),
        ],
        out_specs=pl.BlockSpec((1, D), lambda i: (0, 0)),
        out_shape=jax.ShapeDtypeStruct((1, D), f32),
        scratch_shapes=[pltpu.VMEM((1, D), f32)],
    )(h, W_out_p, b_out_p)

    return out[0, :NCL]


# speedup vs baseline: 1.0004x; 1.0004x over previous
import functools

import jax
import jax.numpy as jnp
from jax import lax
from jax.experimental import pallas as pl
from jax.experimental.pallas import tpu as pltpu
from jax.experimental.pallas import tpu_sc as plsc

NC = 2
NS = 16
CH = 128


def _edge_pass_builder(N_pad, D, TOTALC, K0, K1, SB):
    mesh = plsc.VectorSubcoreMesh(core_axis_name="c", subcore_axis_name="s")
    f32 = jnp.float32
    KMAX = max(K0, K1)

    @functools.partial(
        pl.kernel,
        out_type=[
            jax.ShapeDtypeStruct((N_pad, D), f32),
            jax.ShapeDtypeStruct((N_pad, D), f32),
        ],
        mesh=mesh,
        scratch_types=[
            pltpu.VMEM((KMAX, CH), jnp.int32),
            pltpu.VMEM((KMAX, CH), jnp.int32),
            pltpu.VMEM((CH, D), f32),
            pltpu.VMEM_SHARED((N_pad, D), f32),
            pltpu.SemaphoreType.DMA,
        ],
    )
    def edge_pass(h_hbm, src_hbm, dst_hbm, zero_hbm, out0_hbm, out1_hbm,
                  src_v, dst_v, rows_a, agg_sh, sem_a):
        c = lax.axis_index("c")
        s = lax.axis_index("s")

        @pl.when(s == 0)
        def _():
            pltpu.sync_copy(zero_hbm, agg_sh)

        plsc.subcore_barrier()

        base = jnp.where(c == 0, s * K0, 16 * K0 + s * K1)

        pltpu.sync_copy(src_hbm.at[pl.ds(base, KMAX)], src_v)
        pltpu.sync_copy(dst_hbm.at[pl.ds(base, KMAX)], dst_v)

        def body(j, carry):
            pltpu.async_copy(h_hbm.at[src_v.at[j]], rows_a, sem_a).wait()
            pltpu.sync_copy(rows_a, agg_sh.at[dst_v.at[j]], add=True)
            return carry

        @pl.when(c == 0)
        def _():
            lax.fori_loop(0, K0, body, 0)

        @pl.when(c == 1)
        def _():
            lax.fori_loop(0, K1, body, 0)

        plsc.subcore_barrier()

        @pl.when((s == 0) & (c == 0))
        def _():
            pltpu.sync_copy(agg_sh, out0_hbm)

        @pl.when((s == 0) & (c == 1))
        def _():
            pltpu.sync_copy(agg_sh, out1_hbm)

    return edge_pass


def _deg_builder(N_pad, W, NCHUNK):
    mesh = plsc.VectorSubcoreMesh(core_axis_name="c", subcore_axis_name="s")
    f32 = jnp.float32

    @functools.partial(
        pl.kernel,
        out_type=[
            jax.ShapeDtypeStruct((N_pad, W), f32),
            jax.ShapeDtypeStruct((N_pad, W), f32),
        ],
        mesh=mesh,
        scratch_types=[
            pltpu.VMEM((NCHUNK, CH), jnp.int32),
            pltpu.VMEM((CH, W), f32),
            pltpu.VMEM_SHARED((N_pad, W), f32),
        ],
    )
    def deg_pass(dst_hbm, ones_hbm, zero_hbm, out0_hbm, out1_hbm,
                 dst_v, ones_v, deg_sh):
        c = lax.axis_index("c")
        s = lax.axis_index("s")
        wid = c * NS + s

        @pl.when(s == 0)
        def _():
            pltpu.sync_copy(zero_hbm, deg_sh)

        plsc.subcore_barrier()

        pltpu.sync_copy(dst_hbm.at[wid], dst_v)
        pltpu.sync_copy(ones_hbm, ones_v)

        def body(j, carry):
            pltpu.sync_copy(ones_v, deg_sh.at[dst_v.at[j]], add=True)
            return carry

        lax.fori_loop(0, NCHUNK, body, 0)

        plsc.subcore_barrier()

        @pl.when((s == 0) & (c == 0))
        def _():
            pltpu.sync_copy(deg_sh, out0_hbm)

        @pl.when((s == 0) & (c == 1))
        def _():
            pltpu.sync_copy(deg_sh, out1_hbm)

    return deg_pass


def kernel(x, W_enc, b_enc, W_layers, b_layers, gammas, betas, W_out, b_out,
           edge_index):
    N, D = x.shape
    L = W_layers.shape[0]
    E = edge_index.shape[1]
    NW = NC * NS
    f32 = jnp.float32

    NCHUNK = -(-E // (NW * CH))
    NCHUNK = -(-NCHUNK // 4) * 4
    E_pad = NW * NCHUNK * CH
    N_pad = N + 16

    src = edge_index[0]
    dst = edge_index[1]
    pad = E_pad - E
    src_p = jnp.concatenate([src, jnp.zeros((pad,), jnp.int32)])
    dst_p = jnp.concatenate([dst, jnp.full((pad,), N, jnp.int32)])
    dst3 = dst_p.reshape(NW, NCHUNK, CH)

    zeroND = jnp.zeros((N_pad, D), f32)
    onesW = jnp.ones((CH, D), f32)

    TOTALC = NW * NCHUNK
    K_pair = TOTALC // 16
    K0 = K_pair // 2
    K1 = K_pair - K0
    KMAX = max(K0, K1)
    tail = jnp.zeros((KMAX * CH,), jnp.int32)
    src2 = jnp.concatenate([src_p, tail]).reshape(TOTALC + KMAX, CH)
    dst2 = jnp.concatenate([dst_p, jnp.full((KMAX * CH,), N, jnp.int32)]
                           ).reshape(TOTALC + KMAX, CH)

    edge_pass = _edge_pass_builder(N_pad, D, TOTALC, K0, K1, 0)
    deg_pass = _deg_builder(N_pad, D, NCHUNK)

    RB = 2000 if N % 2000 == 0 else (1000 if N % 1000 == 0 else 8)
    G = N // RB

    def _enc_body(x_ref, w_ref, b_ref, o_ref):
        o_ref[...] = jnp.dot(x_ref[...], w_ref[...],
                             preferred_element_type=f32) + b_ref[...]

    h = pl.pallas_call(
        _enc_body,
        grid=(G,),
        in_specs=[
            pl.BlockSpec((RB, D), lambda i: (i, 0)),
            pl.BlockSpec((D, D), lambda i: (0, 0)),
            pl.BlockSpec((1, D), lambda i: (0, 0)),
        ],
        out_specs=pl.BlockSpec((RB, D), lambda i: (i, 0)),
        out_shape=jax.ShapeDtypeStruct((N, D), f32),
    )(x, W_enc, b_enc[None])

    deg0, deg1 = deg_pass(dst3, onesW, zeroND)

    def _mm_body(p0_ref, p1_ref, d0_ref, d1_ref, w_ref, b_ref,
                 pre_ref, st_ref):
        i = pl.program_id(0)
        deg = jnp.maximum(d0_ref[...][:, :1] + d1_ref[...][:, :1], 1.0)
        agg = (p0_ref[...] + p1_ref[...]) / deg
        pre = jnp.dot(agg, w_ref[...], preferred_element_type=f32) + b_ref[...]
        pre_ref[...] = pre
        s1 = jnp.sum(pre, axis=0, keepdims=True)
        s2 = jnp.sum(pre * pre, axis=0, keepdims=True)
        upd = jnp.concatenate([s1, s2, jnp.zeros((6, pre.shape[1]), f32)],
                              axis=0)
        st_ref[...] = jnp.where(i == 0, upd, st_ref[...] + upd)

    def _bn_body(h_ref, pre_ref, st_ref, g_ref, bt_ref, o_ref):
        mu = st_ref[0:1, :] / float(N)
        ex2 = st_ref[1:2, :] / float(N)
        var = ex2 - mu * mu
        xn = (pre_ref[...] - mu) * lax.rsqrt(var + 1e-5) * g_ref[...] \
            + bt_ref[...]
        o_ref[...] = h_ref[...] + jnp.maximum(xn, 0.0)

    for l in range(L):
        part0, part1 = edge_pass(h, src2, dst2, zeroND)
        pre, stats = pl.pallas_call(
            _mm_body,
            grid=(G,),
            in_specs=[
                pl.BlockSpec((RB, D), lambda i: (i, 0)),
                pl.BlockSpec((RB, D), lambda i: (i, 0)),
                pl.BlockSpec((RB, D), lambda i: (i, 0)),
                pl.BlockSpec((RB, D), lambda i: (i, 0)),
                pl.BlockSpec((D, D), lambda i: (0, 0)),
                pl.BlockSpec((1, D), lambda i: (0, 0)),
            ],
            out_specs=[
                pl.BlockSpec((RB, D), lambda i: (i, 0)),
                pl.BlockSpec((8, D), lambda i: (0, 0)),
            ],
            out_shape=[
                jax.ShapeDtypeStruct((N, D), f32),
                jax.ShapeDtypeStruct((8, D), f32),
            ],
        )(part0, part1, deg0, deg1, W_layers[l], b_layers[l][None])

        h = pl.pallas_call(
            _bn_body,
            grid=(G,),
            in_specs=[
                pl.BlockSpec((RB, D), lambda i: (i, 0)),
                pl.BlockSpec((RB, D), lambda i: (i, 0)),
                pl.BlockSpec((8, D), lambda i: (0, 0)),
                pl.BlockSpec((1, D), lambda i: (0, 0)),
                pl.BlockSpec((1, D), lambda i: (0, 0)),
            ],
            out_specs=pl.BlockSpec((RB, D), lambda i: (i, 0)),
            out_shape=jax.ShapeDtypeStruct((N, D), f32),
        )(h, pre, stats, gammas[l][None], betas[l][None])

    NCL = b_out.shape[0]
    W_out_p = jnp.pad(W_out, ((0, 0), (0, D - NCL)))
    b_out_p = jnp.pad(b_out, (0, D - NCL))[None]

    def _out_body(h_ref, w_ref, b_ref, o_ref, acc_ref):
        i = pl.program_id(0)
        s = jnp.sum(h_ref[...], axis=0, keepdims=True)
        acc_ref[...] = jnp.where(i == 0, s, acc_ref[...] + s)

        @pl.when(i == pl.num_programs(0) - 1)
        def _():
            hg = acc_ref[...] / float(N)
            o_ref[...] = jnp.dot(hg, w_ref[...],
                                 preferred_element_type=f32) + b_ref[...]

    out = pl.pallas_call(
        _out_body,
        grid=(G,),
        in_specs=[
            pl.BlockSpec((RB, D), lambda i: (i, 0)),
            pl.BlockSpec((D, D), lambda i: (0, 0)),
            pl.BlockSpec((1, D), lambda i: (0, 0)),
        ],
        out_specs=pl.BlockSpec((1, D), lambda i: (0, 0)),
        out_shape=jax.ShapeDtypeStruct((1, D), f32),
        scratch_shapes=[pltpu.VMEM((1, D), f32)],
    )(h, W_out_p, b_out_p)

    return out[0, :NCL]


# speedup vs baseline: 1.1631x; 1.1626x over previous
import functools

import jax
import jax.numpy as jnp
from jax import lax
from jax.experimental import pallas as pl
from jax.experimental.pallas import tpu as pltpu
from jax.experimental.pallas import tpu_sc as plsc

NC = 2
NS = 16
CH = 128


def _edge_pass_builder(N_pad, D, TOTALC, K0, K1, SB):
    mesh = plsc.VectorSubcoreMesh(core_axis_name="c", subcore_axis_name="s")
    f32 = jnp.float32
    KMAX = max(K0, K1)

    @functools.partial(
        pl.kernel,
        out_type=[
            jax.ShapeDtypeStruct((N_pad, D), f32),
            jax.ShapeDtypeStruct((N_pad, D), f32),
        ],
        mesh=mesh,
        scratch_types=[
            pltpu.VMEM((KMAX, CH), jnp.int32),
            pltpu.VMEM((KMAX, CH), jnp.int32),
            pltpu.VMEM((CH, D), f32),
            pltpu.VMEM_SHARED((N_pad, D), f32),
            pltpu.SemaphoreType.DMA,
        ],
    )
    def edge_pass(h_hbm, srca_hbm, dsta_hbm, srcb_hbm, dstb_hbm, zero_hbm,
                  out0_hbm, out1_hbm, src_v, dst_v, rows_a, agg_sh, sem_a):
        c = lax.axis_index("c")
        s = lax.axis_index("s")

        @pl.when(s == 0)
        def _():
            pltpu.sync_copy(zero_hbm, agg_sh)

        plsc.subcore_barrier()

        def body(j, carry):
            pltpu.async_copy(h_hbm.at[src_v.at[j]], rows_a, sem_a).wait()
            pltpu.sync_copy(rows_a, agg_sh.at[dst_v.at[j]], add=True)
            return carry

        @pl.when(c == 0)
        def _():
            pltpu.sync_copy(srca_hbm.at[s], src_v.at[pl.ds(0, K0)])
            pltpu.sync_copy(dsta_hbm.at[s], dst_v.at[pl.ds(0, K0)])
            lax.fori_loop(0, K0, body, 0)

        @pl.when(c == 1)
        def _():
            pltpu.sync_copy(srcb_hbm.at[s], src_v.at[pl.ds(0, K1)])
            pltpu.sync_copy(dstb_hbm.at[s], dst_v.at[pl.ds(0, K1)])
            lax.fori_loop(0, K1, body, 0)

        plsc.subcore_barrier()

        @pl.when((s == 0) & (c == 0))
        def _():
            pltpu.sync_copy(agg_sh, out0_hbm)

        @pl.when((s == 0) & (c == 1))
        def _():
            pltpu.sync_copy(agg_sh, out1_hbm)

    return edge_pass


def _deg_builder(N_pad, W, NCHUNK):
    mesh = plsc.VectorSubcoreMesh(core_axis_name="c", subcore_axis_name="s")
    f32 = jnp.float32

    @functools.partial(
        pl.kernel,
        out_type=[
            jax.ShapeDtypeStruct((N_pad, W), f32),
            jax.ShapeDtypeStruct((N_pad, W), f32),
        ],
        mesh=mesh,
        scratch_types=[
            pltpu.VMEM((NCHUNK, CH), jnp.int32),
            pltpu.VMEM((CH, W), f32),
            pltpu.VMEM_SHARED((N_pad, W), f32),
        ],
    )
    def deg_pass(dst_hbm, ones_hbm, zero_hbm, out0_hbm, out1_hbm,
                 dst_v, ones_v, deg_sh):
        c = lax.axis_index("c")
        s = lax.axis_index("s")
        wid = c * NS + s

        @pl.when(s == 0)
        def _():
            pltpu.sync_copy(zero_hbm, deg_sh)

        plsc.subcore_barrier()

        pltpu.sync_copy(dst_hbm.at[wid], dst_v)
        pltpu.sync_copy(ones_hbm, ones_v)

        def body(j, carry):
            pltpu.sync_copy(ones_v, deg_sh.at[dst_v.at[j]], add=True)
            return carry

        lax.fori_loop(0, NCHUNK, body, 0)

        plsc.subcore_barrier()

        @pl.when((s == 0) & (c == 0))
        def _():
            pltpu.sync_copy(deg_sh, out0_hbm)

        @pl.when((s == 0) & (c == 1))
        def _():
            pltpu.sync_copy(deg_sh, out1_hbm)

    return deg_pass


def kernel(x, W_enc, b_enc, W_layers, b_layers, gammas, betas, W_out, b_out,
           edge_index):
    N, D = x.shape
    L = W_layers.shape[0]
    E = edge_index.shape[1]
    NW = NC * NS
    f32 = jnp.float32

    NCHUNK = -(-E // (NW * CH))
    NCHUNK = -(-NCHUNK // 4) * 4
    E_pad = NW * NCHUNK * CH
    N_pad = N + 16

    src = edge_index[0]
    dst = edge_index[1]
    pad = E_pad - E
    src_p = jnp.concatenate([src, jnp.zeros((pad,), jnp.int32)])
    dst_p = jnp.concatenate([dst, jnp.full((pad,), N, jnp.int32)])
    dst3 = dst_p.reshape(NW, NCHUNK, CH)

    zeroND = jnp.zeros((N_pad, D), f32)
    onesW = jnp.ones((CH, D), f32)

    TOTALC = NW * NCHUNK
    K_pair = TOTALC // 16
    K0 = K_pair // 2
    K1 = K_pair - K0
    cut = 16 * K0 * CH
    src_a = src_p[:cut].reshape(16, K0, CH)
    dst_a = dst_p[:cut].reshape(16, K0, CH)
    src_b = src_p[cut:].reshape(16, K1, CH)
    dst_b = dst_p[cut:].reshape(16, K1, CH)

    edge_pass = _edge_pass_builder(N_pad, D, TOTALC, K0, K1, 0)
    deg_pass = _deg_builder(N_pad, D, NCHUNK)

    RB = 2000 if N % 2000 == 0 else (1000 if N % 1000 == 0 else 8)
    G = N // RB

    def _enc_body(x_ref, w_ref, b_ref, o_ref):
        o_ref[...] = jnp.dot(x_ref[...], w_ref[...],
                             preferred_element_type=f32) + b_ref[...]

    h = pl.pallas_call(
        _enc_body,
        grid=(G,),
        in_specs=[
            pl.BlockSpec((RB, D), lambda i: (i, 0)),
            pl.BlockSpec((D, D), lambda i: (0, 0)),
            pl.BlockSpec((1, D), lambda i: (0, 0)),
        ],
        out_specs=pl.BlockSpec((RB, D), lambda i: (i, 0)),
        out_shape=jax.ShapeDtypeStruct((N, D), f32),
    )(x, W_enc, b_enc[None])

    deg0, deg1 = deg_pass(dst3, onesW, zeroND)

    def _mm_body(p0_ref, p1_ref, d0_ref, d1_ref, w_ref, b_ref,
                 pre_ref, st_ref):
        i = pl.program_id(0)
        deg = jnp.maximum(d0_ref[...][:, :1] + d1_ref[...][:, :1], 1.0)
        agg = (p0_ref[...] + p1_ref[...]) / deg
        pre = jnp.dot(agg, w_ref[...], preferred_element_type=f32) + b_ref[...]
        pre_ref[...] = pre
        s1 = jnp.sum(pre, axis=0, keepdims=True)
        s2 = jnp.sum(pre * pre, axis=0, keepdims=True)
        upd = jnp.concatenate([s1, s2, jnp.zeros((6, pre.shape[1]), f32)],
                              axis=0)
        st_ref[...] = jnp.where(i == 0, upd, st_ref[...] + upd)

    def _bn_body(h_ref, pre_ref, st_ref, g_ref, bt_ref, o_ref):
        mu = st_ref[0:1, :] / float(N)
        ex2 = st_ref[1:2, :] / float(N)
        var = ex2 - mu * mu
        xn = (pre_ref[...] - mu) * lax.rsqrt(var + 1e-5) * g_ref[...] \
            + bt_ref[...]
        o_ref[...] = h_ref[...] + jnp.maximum(xn, 0.0)

    for l in range(L):
        part0, part1 = edge_pass(h, src_a, dst_a, src_b, dst_b, zeroND)
        pre, stats = pl.pallas_call(
            _mm_body,
            grid=(G,),
            in_specs=[
                pl.BlockSpec((RB, D), lambda i: (i, 0)),
                pl.BlockSpec((RB, D), lambda i: (i, 0)),
                pl.BlockSpec((RB, D), lambda i: (i, 0)),
                pl.BlockSpec((RB, D), lambda i: (i, 0)),
                pl.BlockSpec((D, D), lambda i: (0, 0)),
                pl.BlockSpec((1, D), lambda i: (0, 0)),
            ],
            out_specs=[
                pl.BlockSpec((RB, D), lambda i: (i, 0)),
                pl.BlockSpec((8, D), lambda i: (0, 0)),
            ],
            out_shape=[
                jax.ShapeDtypeStruct((N, D), f32),
                jax.ShapeDtypeStruct((8, D), f32),
            ],
        )(part0, part1, deg0, deg1, W_layers[l], b_layers[l][None])

        h = pl.pallas_call(
            _bn_body,
            grid=(G,),
            in_specs=[
                pl.BlockSpec((RB, D), lambda i: (i, 0)),
                pl.BlockSpec((RB, D), lambda i: (i, 0)),
                pl.BlockSpec((8, D), lambda i: (0, 0)),
                pl.BlockSpec((1, D), lambda i: (0, 0)),
                pl.BlockSpec((1, D), lambda i: (0, 0)),
            ],
            out_specs=pl.BlockSpec((RB, D), lambda i: (i, 0)),
            out_shape=jax.ShapeDtypeStruct((N, D), f32),
        )(h, pre, stats, gammas[l][None], betas[l][None])

    NCL = b_out.shape[0]
    W_out_p = jnp.pad(W_out, ((0, 0), (0, D - NCL)))
    b_out_p = jnp.pad(b_out, (0, D - NCL))[None]

    def _out_body(h_ref, w_ref, b_ref, o_ref, acc_ref):
        i = pl.program_id(0)
        s = jnp.sum(h_ref[...], axis=0, keepdims=True)
        acc_ref[...] = jnp.where(i == 0, s, acc_ref[...] + s)

        @pl.when(i == pl.num_programs(0) - 1)
        def _():
            hg = acc_ref[...] / float(N)
            o_ref[...] = jnp.dot(hg, w_ref[...],
                                 preferred_element_type=f32) + b_ref[...]

    out = pl.pallas_call(
        _out_body,
        grid=(G,),
        in_specs=[
            pl.BlockSpec((RB, D), lambda i: (i, 0)),
            pl.BlockSpec((D, D), lambda i: (0, 0)),
            pl.BlockSpec((1, D), lambda i: (0, 0)),
        ],
        out_specs=pl.BlockSpec((1, D), lambda i: (0, 0)),
        out_shape=jax.ShapeDtypeStruct((1, D), f32),
        scratch_shapes=[pltpu.VMEM((1, D), f32)],
    )(h, W_out_p, b_out_p)

    return out[0, :NCL]


# speedup vs baseline: 1.5298x; 1.3152x over previous
import functools

import jax
import jax.numpy as jnp
from jax import lax
from jax.experimental import pallas as pl
from jax.experimental.pallas import tpu as pltpu
from jax.experimental.pallas import tpu_sc as plsc

NC = 2
NS = 16
CH = 128


def _edge_pass_builder(N_pad, D, TOTALC, K0, K1, SB):
    mesh = plsc.VectorSubcoreMesh(core_axis_name="c", subcore_axis_name="s")
    f32 = jnp.float32
    KMAX = max(K0, K1)

    @functools.partial(
        pl.kernel,
        out_type=[
            jax.ShapeDtypeStruct((N_pad, D), f32),
            jax.ShapeDtypeStruct((N_pad, D), f32),
        ],
        mesh=mesh,
        scratch_types=[
            pltpu.VMEM((KMAX, CH), jnp.int32),
            pltpu.VMEM((KMAX, CH), jnp.int32),
            pltpu.VMEM((CH, D), f32),
            pltpu.VMEM_SHARED((N_pad, D), f32),
            pltpu.SemaphoreType.DMA,
        ],
    )
    def edge_pass(h_hbm, srca_hbm, dsta_hbm, zero_hbm,
                  out0_hbm, out1_hbm, src_v, dst_v, rows_a, agg_sh, sem_a):
        c = lax.axis_index("c")
        s = lax.axis_index("s")

        @pl.when(s == 0)
        def _():
            pltpu.sync_copy(zero_hbm, agg_sh)

        plsc.subcore_barrier()

        wid = c * NS + s
        pltpu.sync_copy(srca_hbm.at[wid], src_v)
        pltpu.sync_copy(dsta_hbm.at[wid], dst_v)

        def body(j, carry):
            pltpu.async_copy(h_hbm.at[src_v.at[j]], rows_a, sem_a).wait()
            pltpu.sync_copy(rows_a, agg_sh.at[dst_v.at[j]], add=True)
            return carry

        lax.fori_loop(0, K0, body, 0)

        plsc.subcore_barrier()

        @pl.when((s == 0) & (c == 0))
        def _():
            pltpu.sync_copy(agg_sh, out0_hbm)

        @pl.when((s == 0) & (c == 1))
        def _():
            pltpu.sync_copy(agg_sh, out1_hbm)

    return edge_pass


def _deg_builder(N_pad, W, NCHUNK):
    mesh = plsc.VectorSubcoreMesh(core_axis_name="c", subcore_axis_name="s")
    f32 = jnp.float32

    @functools.partial(
        pl.kernel,
        out_type=[
            jax.ShapeDtypeStruct((N_pad, W), f32),
            jax.ShapeDtypeStruct((N_pad, W), f32),
        ],
        mesh=mesh,
        scratch_types=[
            pltpu.VMEM((NCHUNK, CH), jnp.int32),
            pltpu.VMEM((CH, W), f32),
            pltpu.VMEM_SHARED((N_pad, W), f32),
        ],
    )
    def deg_pass(dst_hbm, ones_hbm, zero_hbm, out0_hbm, out1_hbm,
                 dst_v, ones_v, deg_sh):
        c = lax.axis_index("c")
        s = lax.axis_index("s")
        wid = c * NS + s

        @pl.when(s == 0)
        def _():
            pltpu.sync_copy(zero_hbm, deg_sh)

        plsc.subcore_barrier()

        pltpu.sync_copy(dst_hbm.at[wid], dst_v)
        pltpu.sync_copy(ones_hbm, ones_v)

        def body(j, carry):
            pltpu.sync_copy(ones_v, deg_sh.at[dst_v.at[j]], add=True)
            return carry

        lax.fori_loop(0, NCHUNK, body, 0)

        plsc.subcore_barrier()

        @pl.when((s == 0) & (c == 0))
        def _():
            pltpu.sync_copy(deg_sh, out0_hbm)

        @pl.when((s == 0) & (c == 1))
        def _():
            pltpu.sync_copy(deg_sh, out1_hbm)

    return deg_pass


def kernel(x, W_enc, b_enc, W_layers, b_layers, gammas, betas, W_out, b_out,
           edge_index):
    N, D = x.shape
    L = W_layers.shape[0]
    E = edge_index.shape[1]
    NW = NC * NS
    f32 = jnp.float32

    NCHUNK = -(-E // (NW * CH))
    E_pad = NW * NCHUNK * CH
    N_pad = N + 16

    src = edge_index[0]
    dst = edge_index[1]
    pad = E_pad - E
    src_p = jnp.concatenate([src, jnp.zeros((pad,), jnp.int32)])
    dst_p = jnp.concatenate([dst, jnp.full((pad,), N, jnp.int32)])
    dst3 = dst_p.reshape(NW, NCHUNK, CH)

    zeroND = jnp.zeros((N_pad, D), f32)
    onesW = jnp.ones((CH, D), f32)

    TOTALC = NW * NCHUNK
    K_pair = TOTALC // 16
    K0 = K_pair // 2
    K1 = K_pair - K0
    src_a = src_p.reshape(NW, NCHUNK, CH)
    dst_a = dst_p.reshape(NW, NCHUNK, CH)

    edge_pass = _edge_pass_builder(N_pad, D, TOTALC, K0, K1, 0)
    deg_pass = _deg_builder(N_pad, D, NCHUNK)

    RB = 2000 if N % 2000 == 0 else (1000 if N % 1000 == 0 else 8)
    G = N // RB

    def _enc_body(x_ref, w_ref, b_ref, o_ref):
        o_ref[...] = jnp.dot(x_ref[...], w_ref[...],
                             preferred_element_type=f32) + b_ref[...]

    h = pl.pallas_call(
        _enc_body,
        grid=(G,),
        in_specs=[
            pl.BlockSpec((RB, D), lambda i: (i, 0)),
            pl.BlockSpec((D, D), lambda i: (0, 0)),
            pl.BlockSpec((1, D), lambda i: (0, 0)),
        ],
        out_specs=pl.BlockSpec((RB, D), lambda i: (i, 0)),
        out_shape=jax.ShapeDtypeStruct((N, D), f32),
    )(x, W_enc, b_enc[None])

    deg0, deg1 = deg_pass(dst3, onesW, zeroND)

    def _mm_body(p0_ref, p1_ref, d0_ref, d1_ref, w_ref, b_ref,
                 pre_ref, st_ref):
        i = pl.program_id(0)
        deg = jnp.maximum(d0_ref[...][:, :1] + d1_ref[...][:, :1], 1.0)
        agg = (p0_ref[...] + p1_ref[...]) / deg
        pre = jnp.dot(agg, w_ref[...], preferred_element_type=f32) + b_ref[...]
        pre_ref[...] = pre
        s1 = jnp.sum(pre, axis=0, keepdims=True)
        s2 = jnp.sum(pre * pre, axis=0, keepdims=True)
        upd = jnp.concatenate([s1, s2, jnp.zeros((6, pre.shape[1]), f32)],
                              axis=0)
        st_ref[...] = jnp.where(i == 0, upd, st_ref[...] + upd)

    def _bn_body(h_ref, pre_ref, st_ref, g_ref, bt_ref, o_ref):
        mu = st_ref[0:1, :] / float(N)
        ex2 = st_ref[1:2, :] / float(N)
        var = ex2 - mu * mu
        xn = (pre_ref[...] - mu) * lax.rsqrt(var + 1e-5) * g_ref[...] \
            + bt_ref[...]
        o_ref[...] = h_ref[...] + jnp.maximum(xn, 0.0)

    for l in range(L):
        part0, part1 = edge_pass(h, src_a, dst_a, zeroND)
        pre, stats = pl.pallas_call(
            _mm_body,
            grid=(G,),
            in_specs=[
                pl.BlockSpec((RB, D), lambda i: (i, 0)),
                pl.BlockSpec((RB, D), lambda i: (i, 0)),
                pl.BlockSpec((RB, D), lambda i: (i, 0)),
                pl.BlockSpec((RB, D), lambda i: (i, 0)),
                pl.BlockSpec((D, D), lambda i: (0, 0)),
                pl.BlockSpec((1, D), lambda i: (0, 0)),
            ],
            out_specs=[
                pl.BlockSpec((RB, D), lambda i: (i, 0)),
                pl.BlockSpec((8, D), lambda i: (0, 0)),
            ],
            out_shape=[
                jax.ShapeDtypeStruct((N, D), f32),
                jax.ShapeDtypeStruct((8, D), f32),
            ],
        )(part0, part1, deg0, deg1, W_layers[l], b_layers[l][None])

        h = pl.pallas_call(
            _bn_body,
            grid=(G,),
            in_specs=[
                pl.BlockSpec((RB, D), lambda i: (i, 0)),
                pl.BlockSpec((RB, D), lambda i: (i, 0)),
                pl.BlockSpec((8, D), lambda i: (0, 0)),
                pl.BlockSpec((1, D), lambda i: (0, 0)),
                pl.BlockSpec((1, D), lambda i: (0, 0)),
            ],
            out_specs=pl.BlockSpec((RB, D), lambda i: (i, 0)),
            out_shape=jax.ShapeDtypeStruct((N, D), f32),
        )(h, pre, stats, gammas[l][None], betas[l][None])

    NCL = b_out.shape[0]
    W_out_p = jnp.pad(W_out, ((0, 0), (0, D - NCL)))
    b_out_p = jnp.pad(b_out, (0, D - NCL))[None]

    def _out_body(h_ref, w_ref, b_ref, o_ref, acc_ref):
        i = pl.program_id(0)
        s = jnp.sum(h_ref[...], axis=0, keepdims=True)
        acc_ref[...] = jnp.where(i == 0, s, acc_ref[...] + s)

        @pl.when(i == pl.num_programs(0) - 1)
        def _():
            hg = acc_ref[...] / float(N)
            o_ref[...] = jnp.dot(hg, w_ref[...],
                                 preferred_element_type=f32) + b_ref[...]

    out = pl.pallas_call(
        _out_body,
        grid=(G,),
        in_specs=[
            pl.BlockSpec((RB, D), lambda i: (i, 0)),
            pl.BlockSpec((D, D), lambda i: (0, 0)),
            pl.BlockSpec((1, D), lambda i: (0, 0)),
        ],
        out_specs=pl.BlockSpec((1, D), lambda i: (0, 0)),
        out_shape=jax.ShapeDtypeStruct((1, D), f32),
        scratch_shapes=[pltpu.VMEM((1, D), f32)],
    )(h, W_out_p, b_out_p)

    return out[0, :NCL]
